# baseline (speedup 1.0000x reference)
import jax
import jax.numpy as jnp
from jax import lax
from jax.experimental import pallas as pl
from jax.experimental.pallas import tpu as pltpu
from jax.experimental.pallas import tpu_sc as plsc

NUM_FACTORS = 64
BATCH = 16384
NC = 2
NS = 16
NW = NC * NS
BPW = BATCH // NW
CHUNK = 128
NCHUNK = BPW // CHUNK
LANES = 16


def _body(sids_hbm, cids_hbm, stab_hbm, ctab_hbm, out_hbm,
          sidx_v, cidx_v, srows_v, crows_v, outv_v, tpart_v, sem):
    wid = lax.axis_index("s") * NC + lax.axis_index("c")
    base = wid * BPW

    pltpu.sync_copy(sids_hbm.at[pl.ds(wid * NCHUNK, NCHUNK)], sidx_v)
    pltpu.sync_copy(cids_hbm.at[pl.ds(wid * NCHUNK, NCHUNK)], cidx_v)

    copies = []
    for j in range(NCHUNK):
        copies.append(pltpu.async_copy(
            stab_hbm.at[sidx_v.at[j]], srows_v.at[pl.ds(j * CHUNK, CHUNK)], sem))
        copies.append(pltpu.async_copy(
            ctab_hbm.at[cidx_v.at[j]], crows_v.at[pl.ds(j * CHUNK, CHUNK)], sem))
    for cp in copies:
        cp.wait()

    lane_iota = lax.iota(jnp.int32, LANES)

    def block(bi, _):
        rows = bi * LANES + lane_iota
        acc = (plsc.load_gather(srows_v, [rows, jnp.zeros((LANES,), jnp.int32)]) *
               plsc.load_gather(crows_v, [rows, jnp.zeros((LANES,), jnp.int32)]))
        for j in range(1, NUM_FACTORS):
            col = jnp.full((LANES,), j, jnp.int32)
            acc = acc + (plsc.load_gather(srows_v, [rows, col]) *
                         plsc.load_gather(crows_v, [rows, col]))
        outv_v[pl.ds(bi * LANES, LANES)] = acc
        return 0

    lax.fori_loop(0, BPW // LANES, block, 0)

    pltpu.sync_copy(outv_v, out_hbm.at[pl.ds(base, BPW)])


@jax.jit
def _run(sids, cids, stab, ctab):
    mesh = plsc.VectorSubcoreMesh(core_axis_name="c", subcore_axis_name="s",
                                  num_cores=NC, num_subcores=NS)
    return pl.kernel(
        _body,
        out_type=jax.ShapeDtypeStruct((BATCH,), jnp.float32),
        mesh=mesh,
        compiler_params=pltpu.CompilerParams(needs_layout_passes=False,
                                             use_tc_tiling_on_sc=False),
        scratch_types=[
            pltpu.VMEM((NCHUNK, CHUNK), jnp.int32),
            pltpu.VMEM((NCHUNK, CHUNK), jnp.int32),
            pltpu.VMEM((BPW, NUM_FACTORS), jnp.float32),
            pltpu.VMEM((BPW, NUM_FACTORS), jnp.float32),
            pltpu.VMEM((BPW,), jnp.float32),
            pltpu.VMEM((LANES, LANES), jnp.float32),
            pltpu.SemaphoreType.DMA,
        ],
    )(sids, cids, stab, ctab)


def kernel(summoner_ids, champ_ids, summoner_table, champion_table):
    sids = summoner_ids.astype(jnp.int32).reshape(NW * NCHUNK, CHUNK)
    cids = champ_ids.astype(jnp.int32).reshape(NW * NCHUNK, CHUNK)
    return _run(sids, cids, summoner_table, champion_table)
